# baseline (speedup 1.0000x reference)
import functools

import jax
import jax.numpy as jnp
from jax import lax
from jax.experimental import pallas as pl
from jax.experimental.pallas import tpu as pltpu
from jax.experimental.pallas import tpu_sc as plsc

N = 50000
E = 800000
MAX_Z = 95
D3_AUTOANG = 0.52917726
K1C = 16.0
K2C = 4.0 / 3.0
K3C = 4.0
S6 = 1.0
S8 = 0.7875
A1 = 0.4289
A2 = 4.4407
SQRT3 = 1.7320508075688772

NC = 2
NS = 16
NW = NC * NS
L = 16

NP = 50176
NPT = NP // NS
AP = NP // NW
CH = 128
NCH = 196
EW = CH * NCH
EPT = EW * NW

_mesh = plsc.VectorSubcoreMesh(core_axis_name="c", subcore_axis_name="s")
f32 = jnp.float32
i32 = jnp.int32

_atom_out = jax.ShapeDtypeStruct((NP,), f32)


@functools.partial(
    pl.kernel,
    out_type=jax.ShapeDtypeStruct((NC * NP,), f32),
    mesh=_mesh,
    compiler_params=pltpu.CompilerParams(needs_layout_passes=False),
    scratch_types=[
        pltpu.VMEM((96,), f32),
        pltpu.VMEM((CH,), i32),
        pltpu.VMEM((CH,), i32),
        pltpu.VMEM((CH,), f32),
        pltpu.VMEM((CH,), i32),
        pltpu.VMEM((CH,), i32),
        pltpu.VMEM((CH,), f32),
        pltpu.VMEM((NPT,), f32),
        pltpu.VMEM_SHARED((NP,), f32),
        pltpu.SemaphoreType.DMA,
    ],
)
def _k1(z_hbm, rcov_hbm, ii_hbm, jj_hbm, dist_hbm, out_hbm,
        rcov_v, iib, jjb, db, zib, zjb, fbuf, tbuf, cn_sh, sem):
    cid = lax.axis_index("c")
    sid = lax.axis_index("s")
    wid = sid * NC + cid
    pltpu.sync_copy(rcov_hbm, rcov_v)

    zero16 = jnp.zeros((L,), f32)

    def clear(i, c):
        tbuf[pl.ds(i * L, L)] = zero16
        return c

    lax.fori_loop(0, NPT // L, clear, 0)
    pltpu.sync_copy(tbuf, cn_sh.at[pl.ds(sid * NPT, NPT)])
    plsc.subcore_barrier()

    ebase = wid * EW

    def chunk(c, carry):
        base = ebase + c * CH
        d1 = pltpu.async_copy(ii_hbm.at[pl.ds(base, CH)], iib, sem)
        d2 = pltpu.async_copy(jj_hbm.at[pl.ds(base, CH)], jjb, sem)
        d3 = pltpu.async_copy(dist_hbm.at[pl.ds(base, CH)], db, sem)
        d1.wait()
        d2.wait()
        d3.wait()
        g1 = pltpu.async_copy(z_hbm.at[iib], zib, sem)
        g2 = pltpu.async_copy(z_hbm.at[jjb], zjb, sem)
        g1.wait()
        g2.wait()
        for k in range(CH // L):
            sl = pl.ds(k * L, L)
            rc = plsc.load_gather(rcov_v, [zib[sl]]) \
                + plsc.load_gather(rcov_v, [zjb[sl]])
            r = db[sl] * (1.0 / D3_AUTOANG) + 1e-6
            x = K1C * (K2C * rc / r - 1.0)
            e = jnp.exp(-jnp.abs(x))
            num = jnp.where(x >= 0.0, jnp.full((L,), 1.0, f32), e)
            fbuf[sl] = num / (1.0 + e)
        pltpu.sync_copy(fbuf, cn_sh.at[iib], add=True)
        return carry

    lax.fori_loop(0, NCH, chunk, 0)
    plsc.subcore_barrier()
    pltpu.sync_copy(cn_sh.at[pl.ds(sid * NPT, NPT)], tbuf)
    pltpu.sync_copy(tbuf, out_hbm.at[pl.ds(cid * NP + sid * NPT, NPT)])


@functools.partial(
    pl.kernel,
    out_type=[_atom_out] * 5,
    mesh=_mesh,
    compiler_params=pltpu.CompilerParams(needs_layout_passes=False),
    scratch_types=[
        pltpu.VMEM((AP,), i32),
        pltpu.VMEM((AP,), f32),
        pltpu.VMEM((AP,), f32),
        pltpu.VMEM((480,), f32),
        pltpu.VMEM((AP,), f32),
        pltpu.VMEM((AP,), f32),
        pltpu.VMEM((AP,), f32),
        pltpu.VMEM((AP,), f32),
        pltpu.VMEM((AP,), f32),
        pltpu.SemaphoreType.DMA,
    ],
)
def _k2(part_hbm, z_hbm, cnref_hbm,
        o0, o1, o2, o3, o4,
        zw, cn_v, tmp, cnref_v, p0, p1, p2, p3, p4, sem):
    cid = lax.axis_index("c")
    sid = lax.axis_index("s")
    wid = sid * NC + cid
    abase = wid * AP
    pltpu.sync_copy(z_hbm.at[pl.ds(abase, AP)], zw)
    pltpu.sync_copy(cnref_hbm, cnref_v)
    pltpu.sync_copy(part_hbm.at[pl.ds(abase, AP)], cn_v)
    pltpu.sync_copy(part_hbm.at[pl.ds(NP + abase, AP)], tmp)

    pv = [p0, p1, p2, p3, p4]

    def grp(i, c):
        sl = pl.ds(i * L, L)
        cn = cn_v[sl] + tmp[sl]
        z5 = zw[sl] * 5
        li = []
        for q in range(5):
            refq = plsc.load_gather(cnref_v, [z5 + q])
            d = cn - refq
            li.append(-K3C * d * d)
        m = jnp.maximum(jnp.maximum(jnp.maximum(li[0], li[1]),
                                    jnp.maximum(li[2], li[3])), li[4])
        u = [jnp.exp(t - m) for t in li]
        s = u[0] + u[1] + u[2] + u[3] + u[4]
        inv = 1.0 / s
        for q in range(5):
            pv[q][sl] = u[q] * inv
        return c

    lax.fori_loop(0, AP // L, grp, 0)
    for q, o in enumerate((o0, o1, o2, o3, o4)):
        pltpu.sync_copy(pv[q], o.at[pl.ds(abase, AP)])


@functools.partial(
    pl.kernel,
    out_type=jax.ShapeDtypeStruct((NC * NP,), f32),
    mesh=_mesh,
    compiler_params=pltpu.CompilerParams(needs_layout_passes=False),
    scratch_types=[pltpu.VMEM((96,), f32)]
    + ([pltpu.VMEM((CH,), i32)] * 5
       + [pltpu.VMEM((CH,), f32)] * 10) * 2
    + [
        pltpu.VMEM((CH, 128), f32),
        pltpu.VMEM((NPT,), f32),
        pltpu.VMEM_SHARED((NP,), f32),
        pltpu.SemaphoreType.DMA,
    ],
)
def _k3(a0, a1, a2, a3, c6_hbm, z_hbm, r4s_hbm, ii_hbm, jj_hbm,
        dist_hbm, out_hbm, r4s_v, *rest):
    setA = rest[0:15]
    setB = rest[15:30]
    cb = rest[30]
    tbuf = rest[31]
    en_sh = rest[32]
    sem = rest[33]
    cid = lax.axis_index("c")
    sid = lax.axis_index("s")
    wid = sid * NC + cid
    pltpu.sync_copy(r4s_hbm, r4s_v)

    zero16 = jnp.zeros((L,), f32)
    one16 = jnp.full((L,), 1.0, f32)

    def clear(i, c):
        tbuf[pl.ds(i * L, L)] = zero16
        return c

    lax.fori_loop(0, NPT // L, clear, 0)
    pltpu.sync_copy(tbuf, en_sh.at[pl.ds(sid * NPT, NPT)])
    plsc.subcore_barrier()

    ebase = wid * EW
    lane = lax.iota(i32, L)

    def issue_loads(base, s):
        iib, jjb, db = s[0], s[1], s[5]
        return [pltpu.async_copy(ii_hbm.at[pl.ds(base, CH)], iib, sem),
                pltpu.async_copy(jj_hbm.at[pl.ds(base, CH)], jjb, sem),
                pltpu.async_copy(dist_hbm.at[pl.ds(base, CH)], db, sem)]

    def issue_gathers(s):
        iib, jjb, zib, zjb = s[0], s[1], s[2], s[3]
        av = [a0, a1, a2, a3]
        gz = [pltpu.async_copy(z_hbm.at[iib], zib, sem),
              pltpu.async_copy(z_hbm.at[jjb], zjb, sem)]
        gp = [pltpu.async_copy(av[q].at[iib], s[6 + q], sem)
              for q in range(4)]
        gq = [pltpu.async_copy(av[q].at[jjb], s[10 + q], sem)
              for q in range(4)]
        return gz, gp + gq

    def pair_and_w(s):
        zib, zjb, pairb, db, ebuf = s[2], s[3], s[4], s[5], s[14]
        for k in range(CH // L):
            sl = pl.ds(k * L, L)
            pairb[sl] = zib[sl] * MAX_Z + zjb[sl]
            s4i = plsc.load_gather(r4s_v, [zib[sl]])
            s4j = plsc.load_gather(r4s_v, [zjb[sl]])
            r = db[sl] * (1.0 / D3_AUTOANG) + 1e-6
            r2 = r * r
            r6 = r2 * r2 * r2
            r8 = r6 * r2
            ss = s4i * s4j
            qq = 3.0 * ss * ss
            r0 = (A1 * SQRT3) * ss + A2
            r02 = r0 * r0
            r06 = r02 * r02 * r02
            r08 = r06 * r02
            ebuf[sl] = (-0.5 * S6) / (r6 + r06) \
                + ((-0.5 * S8) * qq) / (r8 + r08)
        return pltpu.async_copy(c6_hbm.at[pairb], cb, sem)

    def c6_scale_scatter(s):
        iib, ebuf = s[0], s[14]
        for k in range(CH // L):
            sl = pl.ds(k * L, L)
            eids = k * L + lane
            pi = [s[6 + q][sl] for q in range(4)]
            pj = [s[10 + q][sl] for q in range(4)]
            pi.append(one16 - pi[0] - pi[1] - pi[2] - pi[3])
            pj.append(one16 - pj[0] - pj[1] - pj[2] - pj[3])
            c6 = jnp.zeros((L,), f32)
            for a in range(5):
                rowacc = jnp.zeros((L,), f32)
                for b in range(5):
                    cab = plsc.load_gather(
                        cb, [eids, jnp.full((L,), a * 5 + b, i32)])
                    rowacc = rowacc + pj[b] * cab
                c6 = c6 + pi[a] * rowacc
            ebuf[sl] = c6 * ebuf[sl]
        pltpu.sync_copy(ebuf, en_sh.at[iib], add=True)

    def half(c2, carry):
        baseA = ebase + (c2 * 2) * CH
        dA = issue_loads(baseA, setA)
        dB = issue_loads(baseA + CH, setB)
        for d in dA:
            d.wait()
        gzA, gpA = issue_gathers(setA)
        for g in gzA:
            g.wait()
        g3A = pair_and_w(setA)
        for d in dB:
            d.wait()
        gzB, gpB = issue_gathers(setB)
        for g in gpA:
            g.wait()
        g3A.wait()
        c6_scale_scatter(setA)
        for g in gzB:
            g.wait()
        g3B = pair_and_w(setB)
        for g in gpB:
            g.wait()
        g3B.wait()
        c6_scale_scatter(setB)
        return carry

    lax.fori_loop(0, NCH // 2, half, 0)
    plsc.subcore_barrier()
    pltpu.sync_copy(en_sh.at[pl.ds(sid * NPT, NPT)], tbuf)
    pltpu.sync_copy(tbuf, out_hbm.at[pl.ds(cid * NP + sid * NPT, NPT)])


@functools.partial(
    pl.kernel,
    out_type=jax.ShapeDtypeStruct((NP,), f32),
    mesh=_mesh,
    compiler_params=pltpu.CompilerParams(needs_layout_passes=False),
    scratch_types=[
        pltpu.VMEM((AP,), f32),
        pltpu.VMEM((AP,), f32),
        pltpu.SemaphoreType.DMA,
    ],
)
def _k4(part_hbm, out_hbm, s_v, tmp, sem):
    cid = lax.axis_index("c")
    sid = lax.axis_index("s")
    wid = sid * NC + cid
    abase = wid * AP
    pltpu.sync_copy(part_hbm.at[pl.ds(abase, AP)], s_v)
    pltpu.sync_copy(part_hbm.at[pl.ds(NP + abase, AP)], tmp)

    def add(i, c):
        sl = pl.ds(i * L, L)
        s_v[sl] = s_v[sl] + tmp[sl]
        return c

    lax.fori_loop(0, AP // L, add, 0)
    pltpu.sync_copy(s_v, out_hbm.at[pl.ds(abase, AP)])


def kernel(Z, edge_dist, edge_index, rcov, r4r2, cn_ref, c6_ref):
    Zp = jnp.concatenate([Z.astype(i32), jnp.zeros((NP - N,), i32)])
    ii = jnp.concatenate(
        [edge_index[0].astype(i32), jnp.full((EPT - E,), N, i32)])
    jj = jnp.concatenate(
        [edge_index[1].astype(i32), jnp.full((EPT - E,), N, i32)])
    dist = jnp.concatenate(
        [edge_dist.astype(f32), jnp.ones((EPT - E,), f32)])
    rcov96 = jnp.pad(rcov.astype(f32), (0, 96 - MAX_Z))
    r4s96 = jnp.pad(jnp.sqrt(r4r2.astype(f32)), (0, 96 - MAX_Z))
    cnref480 = jnp.pad(cn_ref.astype(f32).reshape(-1), (0, 5))
    c6p = jnp.pad(c6_ref.astype(f32).reshape(MAX_Z * MAX_Z, 25),
                  ((0, 0), (0, 103)))

    cnpart = _k1(Zp, rcov96, ii, jj, dist)
    p5 = _k2(cnpart, Zp, cnref480)
    enpart = _k3(p5[0], p5[1], p5[2], p5[3],
                 c6p, Zp, r4s96, ii, jj, dist)
    out = _k4(enpart)
    return out[:N]


# speedup vs baseline: 1.0976x; 1.0976x over previous
import functools

import jax
import jax.numpy as jnp
from jax import lax
from jax.experimental import pallas as pl
from jax.experimental.pallas import tpu as pltpu
from jax.experimental.pallas import tpu_sc as plsc

N = 50000
E = 800000
MAX_Z = 95
D3_AUTOANG = 0.52917726
K1C = 16.0
K2C = 4.0 / 3.0
K3C = 4.0
S6 = 1.0
S8 = 0.7875
A1 = 0.4289
A2 = 4.4407
SQRT3 = 1.7320508075688772

NC = 2
NS = 16
NW = NC * NS
L = 16

NP = 50176
NPT = NP // NS
AP = NP // NW
CH = 128
NCH = 196
EW = CH * NCH
EPT = EW * NW

_mesh = plsc.VectorSubcoreMesh(core_axis_name="c", subcore_axis_name="s")
f32 = jnp.float32
i32 = jnp.int32

_atom_out = jax.ShapeDtypeStruct((NP,), f32)


@functools.partial(
    pl.kernel,
    out_type=jax.ShapeDtypeStruct((NC * NP,), f32),
    mesh=_mesh,
    compiler_params=pltpu.CompilerParams(needs_layout_passes=False),
    scratch_types=[
        pltpu.VMEM((96,), f32),
        pltpu.VMEM((CH,), i32),
        pltpu.VMEM((CH,), i32),
        pltpu.VMEM((CH,), f32),
        pltpu.VMEM((CH,), i32),
        pltpu.VMEM((CH,), i32),
        pltpu.VMEM((CH,), f32),
        pltpu.VMEM((NPT,), f32),
        pltpu.VMEM_SHARED((NP,), f32),
        pltpu.SemaphoreType.DMA,
    ],
)
def _k1(z_hbm, rcov_hbm, ii_hbm, jj_hbm, dist_hbm, out_hbm,
        rcov_v, iib, jjb, db, zib, zjb, fbuf, tbuf, cn_sh, sem):
    cid = lax.axis_index("c")
    sid = lax.axis_index("s")
    wid = sid * NC + cid
    pltpu.sync_copy(rcov_hbm, rcov_v)

    zero16 = jnp.zeros((L,), f32)

    def clear(i, c):
        tbuf[pl.ds(i * L, L)] = zero16
        return c

    lax.fori_loop(0, NPT // L, clear, 0)
    pltpu.sync_copy(tbuf, cn_sh.at[pl.ds(sid * NPT, NPT)])
    plsc.subcore_barrier()

    ebase = wid * EW

    def chunk(c, carry):
        base = ebase + c * CH
        d1 = pltpu.async_copy(ii_hbm.at[pl.ds(base, CH)], iib, sem)
        d2 = pltpu.async_copy(jj_hbm.at[pl.ds(base, CH)], jjb, sem)
        d3 = pltpu.async_copy(dist_hbm.at[pl.ds(base, CH)], db, sem)
        d1.wait()
        d2.wait()
        d3.wait()
        g1 = pltpu.async_copy(z_hbm.at[iib], zib, sem)
        g2 = pltpu.async_copy(z_hbm.at[jjb], zjb, sem)
        g1.wait()
        g2.wait()
        for k in range(CH // L):
            sl = pl.ds(k * L, L)
            rc = plsc.load_gather(rcov_v, [zib[sl]]) \
                + plsc.load_gather(rcov_v, [zjb[sl]])
            r = db[sl] * (1.0 / D3_AUTOANG) + 1e-6
            x = K1C * (K2C * rc / r - 1.0)
            e = jnp.exp(-jnp.abs(x))
            num = jnp.where(x >= 0.0, jnp.full((L,), 1.0, f32), e)
            fbuf[sl] = num / (1.0 + e)
        pltpu.sync_copy(fbuf, cn_sh.at[iib], add=True)
        return carry

    lax.fori_loop(0, NCH, chunk, 0)
    plsc.subcore_barrier()
    pltpu.sync_copy(cn_sh.at[pl.ds(sid * NPT, NPT)], tbuf)
    pltpu.sync_copy(tbuf, out_hbm.at[pl.ds(cid * NP + sid * NPT, NPT)])


@functools.partial(
    pl.kernel,
    out_type=[_atom_out] * 5,
    mesh=_mesh,
    compiler_params=pltpu.CompilerParams(needs_layout_passes=False),
    scratch_types=[
        pltpu.VMEM((AP,), i32),
        pltpu.VMEM((AP,), f32),
        pltpu.VMEM((AP,), f32),
        pltpu.VMEM((480,), f32),
        pltpu.VMEM((AP,), f32),
        pltpu.VMEM((AP,), f32),
        pltpu.VMEM((AP,), f32),
        pltpu.VMEM((AP,), f32),
        pltpu.VMEM((AP,), f32),
        pltpu.SemaphoreType.DMA,
    ],
)
def _k2(part_hbm, z_hbm, cnref_hbm,
        o0, o1, o2, o3, o4,
        zw, cn_v, tmp, cnref_v, p0, p1, p2, p3, p4, sem):
    cid = lax.axis_index("c")
    sid = lax.axis_index("s")
    wid = sid * NC + cid
    abase = wid * AP
    pltpu.sync_copy(z_hbm.at[pl.ds(abase, AP)], zw)
    pltpu.sync_copy(cnref_hbm, cnref_v)
    pltpu.sync_copy(part_hbm.at[pl.ds(abase, AP)], cn_v)
    pltpu.sync_copy(part_hbm.at[pl.ds(NP + abase, AP)], tmp)

    pv = [p0, p1, p2, p3, p4]

    def grp(i, c):
        sl = pl.ds(i * L, L)
        cn = cn_v[sl] + tmp[sl]
        z5 = zw[sl] * 5
        li = []
        for q in range(5):
            refq = plsc.load_gather(cnref_v, [z5 + q])
            d = cn - refq
            li.append(-K3C * d * d)
        m = jnp.maximum(jnp.maximum(jnp.maximum(li[0], li[1]),
                                    jnp.maximum(li[2], li[3])), li[4])
        u = [jnp.exp(t - m) for t in li]
        s = u[0] + u[1] + u[2] + u[3] + u[4]
        inv = 1.0 / s
        for q in range(5):
            pv[q][sl] = u[q] * inv
        return c

    lax.fori_loop(0, AP // L, grp, 0)
    for q, o in enumerate((o0, o1, o2, o3, o4)):
        pltpu.sync_copy(pv[q], o.at[pl.ds(abase, AP)])


@functools.partial(
    pl.kernel,
    out_type=jax.ShapeDtypeStruct((NC * NP,), f32),
    mesh=_mesh,
    compiler_params=pltpu.CompilerParams(needs_layout_passes=False),
    scratch_types=[pltpu.VMEM((96,), f32)]
    + ([pltpu.VMEM((CH,), i32)] * 5
       + [pltpu.VMEM((CH,), f32)] * 10) * 2
    + [
        pltpu.VMEM((CH, 128), f32),
        pltpu.VMEM((NPT,), f32),
        pltpu.VMEM_SHARED((NP,), f32),
    ] + [pltpu.SemaphoreType.DMA] * 8,
)
def _k3(a0, a1, a2, a3, c6_hbm, z_hbm, r4s_hbm, ii_hbm, jj_hbm,
        dist_hbm, out_hbm, r4s_v, *rest):
    setA = rest[0:15]
    setB = rest[15:30]
    cb = rest[30]
    tbuf = rest[31]
    en_sh = rest[32]
    semA = rest[33:37]
    semB = rest[37:41]
    cid = lax.axis_index("c")
    sid = lax.axis_index("s")
    wid = sid * NC + cid
    pltpu.sync_copy(r4s_hbm, r4s_v)

    zero16 = jnp.zeros((L,), f32)
    one16 = jnp.full((L,), 1.0, f32)

    def clear(i, c):
        tbuf[pl.ds(i * L, L)] = zero16
        return c

    lax.fori_loop(0, NPT // L, clear, 0)
    pltpu.sync_copy(tbuf, en_sh.at[pl.ds(sid * NPT, NPT)])
    plsc.subcore_barrier()

    ebase = wid * EW
    lane = lax.iota(i32, L)

    def issue_loads(base, s, sg):
        iib, jjb, db = s[0], s[1], s[5]
        return [pltpu.async_copy(ii_hbm.at[pl.ds(base, CH)], iib, sg[0]),
                pltpu.async_copy(jj_hbm.at[pl.ds(base, CH)], jjb, sg[0]),
                pltpu.async_copy(dist_hbm.at[pl.ds(base, CH)], db, sg[0])]

    def issue_gathers(s, sg):
        iib, jjb, zib, zjb = s[0], s[1], s[2], s[3]
        av = [a0, a1, a2, a3]
        gz = [pltpu.async_copy(z_hbm.at[iib], zib, sg[1]),
              pltpu.async_copy(z_hbm.at[jjb], zjb, sg[1])]
        gp = [pltpu.async_copy(av[q].at[iib], s[6 + q], sg[2])
              for q in range(4)]
        gq = [pltpu.async_copy(av[q].at[jjb], s[10 + q], sg[2])
              for q in range(4)]
        return gz, gp + gq

    def pair_and_w(s, sg):
        zib, zjb, pairb, db, ebuf = s[2], s[3], s[4], s[5], s[14]
        for k in range(CH // L):
            sl = pl.ds(k * L, L)
            pairb[sl] = zib[sl] * MAX_Z + zjb[sl]
            s4i = plsc.load_gather(r4s_v, [zib[sl]])
            s4j = plsc.load_gather(r4s_v, [zjb[sl]])
            r = db[sl] * (1.0 / D3_AUTOANG) + 1e-6
            r2 = r * r
            r6 = r2 * r2 * r2
            r8 = r6 * r2
            ss = s4i * s4j
            qq = 3.0 * ss * ss
            r0 = (A1 * SQRT3) * ss + A2
            r02 = r0 * r0
            r06 = r02 * r02 * r02
            r08 = r06 * r02
            ebuf[sl] = (-0.5 * S6) / (r6 + r06) \
                + ((-0.5 * S8) * qq) / (r8 + r08)
        return pltpu.async_copy(c6_hbm.at[pairb], cb, sg[3])

    def c6_scale_scatter(s):
        iib, ebuf = s[0], s[14]
        for k in range(CH // L):
            sl = pl.ds(k * L, L)
            eids = k * L + lane
            pi = [s[6 + q][sl] for q in range(4)]
            pj = [s[10 + q][sl] for q in range(4)]
            pi.append(one16 - pi[0] - pi[1] - pi[2] - pi[3])
            pj.append(one16 - pj[0] - pj[1] - pj[2] - pj[3])
            c6 = jnp.zeros((L,), f32)
            for a in range(5):
                rowacc = jnp.zeros((L,), f32)
                for b in range(5):
                    cab = plsc.load_gather(
                        cb, [eids, jnp.full((L,), a * 5 + b, i32)])
                    rowacc = rowacc + pj[b] * cab
                c6 = c6 + pi[a] * rowacc
            ebuf[sl] = c6 * ebuf[sl]
        pltpu.sync_copy(ebuf, en_sh.at[iib], add=True)

    def half(c2, carry):
        baseA = ebase + (c2 * 2) * CH
        dA = issue_loads(baseA, setA, semA)
        dB = issue_loads(baseA + CH, setB, semB)
        for d in dA:
            d.wait()
        gzA, gpA = issue_gathers(setA, semA)
        for g in gzA:
            g.wait()
        g3A = pair_and_w(setA, semA)
        for d in dB:
            d.wait()
        gzB, gpB = issue_gathers(setB, semB)
        for g in gpA:
            g.wait()
        g3A.wait()
        c6_scale_scatter(setA)
        for g in gzB:
            g.wait()
        g3B = pair_and_w(setB, semB)
        for g in gpB:
            g.wait()
        g3B.wait()
        c6_scale_scatter(setB)
        return carry

    lax.fori_loop(0, NCH // 2, half, 0)
    plsc.subcore_barrier()
    pltpu.sync_copy(en_sh.at[pl.ds(sid * NPT, NPT)], tbuf)
    pltpu.sync_copy(tbuf, out_hbm.at[pl.ds(cid * NP + sid * NPT, NPT)])


@functools.partial(
    pl.kernel,
    out_type=jax.ShapeDtypeStruct((NP,), f32),
    mesh=_mesh,
    compiler_params=pltpu.CompilerParams(needs_layout_passes=False),
    scratch_types=[
        pltpu.VMEM((AP,), f32),
        pltpu.VMEM((AP,), f32),
        pltpu.SemaphoreType.DMA,
    ],
)
def _k4(part_hbm, out_hbm, s_v, tmp, sem):
    cid = lax.axis_index("c")
    sid = lax.axis_index("s")
    wid = sid * NC + cid
    abase = wid * AP
    pltpu.sync_copy(part_hbm.at[pl.ds(abase, AP)], s_v)
    pltpu.sync_copy(part_hbm.at[pl.ds(NP + abase, AP)], tmp)

    def add(i, c):
        sl = pl.ds(i * L, L)
        s_v[sl] = s_v[sl] + tmp[sl]
        return c

    lax.fori_loop(0, AP // L, add, 0)
    pltpu.sync_copy(s_v, out_hbm.at[pl.ds(abase, AP)])


def kernel(Z, edge_dist, edge_index, rcov, r4r2, cn_ref, c6_ref):
    Zp = jnp.concatenate([Z.astype(i32), jnp.zeros((NP - N,), i32)])
    ii = jnp.concatenate(
        [edge_index[0].astype(i32), jnp.full((EPT - E,), N, i32)])
    jj = jnp.concatenate(
        [edge_index[1].astype(i32), jnp.full((EPT - E,), N, i32)])
    dist = jnp.concatenate(
        [edge_dist.astype(f32), jnp.ones((EPT - E,), f32)])
    rcov96 = jnp.pad(rcov.astype(f32), (0, 96 - MAX_Z))
    r4s96 = jnp.pad(jnp.sqrt(r4r2.astype(f32)), (0, 96 - MAX_Z))
    cnref480 = jnp.pad(cn_ref.astype(f32).reshape(-1), (0, 5))
    c6p = jnp.pad(c6_ref.astype(f32).reshape(MAX_Z * MAX_Z, 25),
                  ((0, 0), (0, 103)))

    cnpart = _k1(Zp, rcov96, ii, jj, dist)
    p5 = _k2(cnpart, Zp, cnref480)
    enpart = _k3(p5[0], p5[1], p5[2], p5[3],
                 c6p, Zp, r4s96, ii, jj, dist)
    out = _k4(enpart)
    return out[:N]


# speedup vs baseline: 1.2064x; 1.0991x over previous
import functools

import jax
import jax.numpy as jnp
from jax import lax
from jax.experimental import pallas as pl
from jax.experimental.pallas import tpu as pltpu
from jax.experimental.pallas import tpu_sc as plsc

N = 50000
E = 800000
MAX_Z = 95
D3_AUTOANG = 0.52917726
K1C = 16.0
K2C = 4.0 / 3.0
K3C = 4.0
S6 = 1.0
S8 = 0.7875
A1 = 0.4289
A2 = 4.4407
SQRT3 = 1.7320508075688772

NC = 2
NS = 16
NW = NC * NS
L = 16

NP = 50176
NPT = NP // NS
AP = NP // NW
CH = 128
NCH = 196
EW = CH * NCH
EPT = EW * NW

_mesh = plsc.VectorSubcoreMesh(core_axis_name="c", subcore_axis_name="s")
f32 = jnp.float32
i32 = jnp.int32

_atom_out = jax.ShapeDtypeStruct((NP,), f32)


@functools.partial(
    pl.kernel,
    out_type=jax.ShapeDtypeStruct((NC * NP,), f32),
    mesh=_mesh,
    compiler_params=pltpu.CompilerParams(needs_layout_passes=False),
    scratch_types=[pltpu.VMEM((96,), f32)]
    + ([pltpu.VMEM((CH,), i32)] * 4
       + [pltpu.VMEM((CH,), f32)] * 2) * 2
    + [
        pltpu.VMEM((NPT,), f32),
        pltpu.VMEM_SHARED((NP,), f32),
    ] + [pltpu.SemaphoreType.DMA] * 4,
)
def _k1(z_hbm, rcov_hbm, ii_hbm, jj_hbm, dist_hbm, out_hbm,
        rcov_v, *rest):
    setA = rest[0:6]
    setB = rest[6:12]
    tbuf = rest[12]
    cn_sh = rest[13]
    semA = rest[14:16]
    semB = rest[16:18]
    cid = lax.axis_index("c")
    sid = lax.axis_index("s")
    wid = sid * NC + cid
    pltpu.sync_copy(rcov_hbm, rcov_v)

    zero16 = jnp.zeros((L,), f32)

    def clear(i, c):
        tbuf[pl.ds(i * L, L)] = zero16
        return c

    lax.fori_loop(0, NPT // L, clear, 0)
    pltpu.sync_copy(tbuf, cn_sh.at[pl.ds(sid * NPT, NPT)])
    plsc.subcore_barrier()

    ebase = wid * EW

    def issue_loads(base, s, sg):
        return [pltpu.async_copy(ii_hbm.at[pl.ds(base, CH)], s[0], sg[0]),
                pltpu.async_copy(jj_hbm.at[pl.ds(base, CH)], s[1], sg[0]),
                pltpu.async_copy(dist_hbm.at[pl.ds(base, CH)], s[4], sg[0])]

    def issue_z(s, sg):
        return [pltpu.async_copy(z_hbm.at[s[0]], s[2], sg[1]),
                pltpu.async_copy(z_hbm.at[s[1]], s[3], sg[1])]

    def cn_scatter(s):
        iib, zib, zjb, db, fbuf = s[0], s[2], s[3], s[4], s[5]
        for k in range(CH // L):
            sl = pl.ds(k * L, L)
            rc = plsc.load_gather(rcov_v, [zib[sl]]) \
                + plsc.load_gather(rcov_v, [zjb[sl]])
            r = db[sl] * (1.0 / D3_AUTOANG) + 1e-6
            x = K1C * (K2C * rc / r - 1.0)
            e = jnp.exp(-jnp.abs(x))
            num = jnp.where(x >= 0.0, jnp.full((L,), 1.0, f32), e)
            fbuf[sl] = num / (1.0 + e)
        pltpu.sync_copy(fbuf, cn_sh.at[iib], add=True)

    def half(c2, carry):
        baseA = ebase + (c2 * 2) * CH
        dA = issue_loads(baseA, setA, semA)
        dB = issue_loads(baseA + CH, setB, semB)
        for d in dA:
            d.wait()
        gA = issue_z(setA, semA)
        for d in dB:
            d.wait()
        gB = issue_z(setB, semB)
        for g in gA:
            g.wait()
        cn_scatter(setA)
        for g in gB:
            g.wait()
        cn_scatter(setB)
        return carry

    lax.fori_loop(0, NCH // 2, half, 0)
    plsc.subcore_barrier()
    pltpu.sync_copy(cn_sh.at[pl.ds(sid * NPT, NPT)], tbuf)
    pltpu.sync_copy(tbuf, out_hbm.at[pl.ds(cid * NP + sid * NPT, NPT)])


@functools.partial(
    pl.kernel,
    out_type=[_atom_out] * 5,
    mesh=_mesh,
    compiler_params=pltpu.CompilerParams(needs_layout_passes=False),
    scratch_types=[
        pltpu.VMEM((AP,), i32),
        pltpu.VMEM((AP,), f32),
        pltpu.VMEM((AP,), f32),
        pltpu.VMEM((480,), f32),
        pltpu.VMEM((AP,), f32),
        pltpu.VMEM((AP,), f32),
        pltpu.VMEM((AP,), f32),
        pltpu.VMEM((AP,), f32),
        pltpu.VMEM((AP,), f32),
        pltpu.SemaphoreType.DMA,
    ],
)
def _k2(part_hbm, z_hbm, cnref_hbm,
        o0, o1, o2, o3, o4,
        zw, cn_v, tmp, cnref_v, p0, p1, p2, p3, p4, sem):
    cid = lax.axis_index("c")
    sid = lax.axis_index("s")
    wid = sid * NC + cid
    abase = wid * AP
    pltpu.sync_copy(z_hbm.at[pl.ds(abase, AP)], zw)
    pltpu.sync_copy(cnref_hbm, cnref_v)
    pltpu.sync_copy(part_hbm.at[pl.ds(abase, AP)], cn_v)
    pltpu.sync_copy(part_hbm.at[pl.ds(NP + abase, AP)], tmp)

    pv = [p0, p1, p2, p3, p4]

    def grp(i, c):
        sl = pl.ds(i * L, L)
        cn = cn_v[sl] + tmp[sl]
        z5 = zw[sl] * 5
        li = []
        for q in range(5):
            refq = plsc.load_gather(cnref_v, [z5 + q])
            d = cn - refq
            li.append(-K3C * d * d)
        m = jnp.maximum(jnp.maximum(jnp.maximum(li[0], li[1]),
                                    jnp.maximum(li[2], li[3])), li[4])
        u = [jnp.exp(t - m) for t in li]
        s = u[0] + u[1] + u[2] + u[3] + u[4]
        inv = 1.0 / s
        for q in range(5):
            pv[q][sl] = u[q] * inv
        return c

    lax.fori_loop(0, AP // L, grp, 0)
    for q, o in enumerate((o0, o1, o2, o3, o4)):
        pltpu.sync_copy(pv[q], o.at[pl.ds(abase, AP)])


@functools.partial(
    pl.kernel,
    out_type=jax.ShapeDtypeStruct((NC * NP,), f32),
    mesh=_mesh,
    compiler_params=pltpu.CompilerParams(needs_layout_passes=False),
    scratch_types=[pltpu.VMEM((96,), f32)]
    + ([pltpu.VMEM((CH,), i32)] * 5
       + [pltpu.VMEM((CH,), f32)] * 10) * 2
    + [
        pltpu.VMEM((CH, 128), f32),
        pltpu.VMEM((NPT,), f32),
        pltpu.VMEM_SHARED((NP,), f32),
    ] + [pltpu.SemaphoreType.DMA] * 8,
)
def _k3(a0, a1, a2, a3, c6_hbm, z_hbm, r4s_hbm, ii_hbm, jj_hbm,
        dist_hbm, out_hbm, r4s_v, *rest):
    setA = rest[0:15]
    setB = rest[15:30]
    cb = rest[30]
    tbuf = rest[31]
    en_sh = rest[32]
    semA = rest[33:37]
    semB = rest[37:41]
    cid = lax.axis_index("c")
    sid = lax.axis_index("s")
    wid = sid * NC + cid
    pltpu.sync_copy(r4s_hbm, r4s_v)

    zero16 = jnp.zeros((L,), f32)
    one16 = jnp.full((L,), 1.0, f32)

    def clear(i, c):
        tbuf[pl.ds(i * L, L)] = zero16
        return c

    lax.fori_loop(0, NPT // L, clear, 0)
    pltpu.sync_copy(tbuf, en_sh.at[pl.ds(sid * NPT, NPT)])
    plsc.subcore_barrier()

    ebase = wid * EW
    lane = lax.iota(i32, L)

    def issue_loads(base, s, sg):
        iib, jjb, db = s[0], s[1], s[5]
        return [pltpu.async_copy(ii_hbm.at[pl.ds(base, CH)], iib, sg[0]),
                pltpu.async_copy(jj_hbm.at[pl.ds(base, CH)], jjb, sg[0]),
                pltpu.async_copy(dist_hbm.at[pl.ds(base, CH)], db, sg[0])]

    def issue_gathers(s, sg):
        iib, jjb, zib, zjb = s[0], s[1], s[2], s[3]
        av = [a0, a1, a2, a3]
        gz = [pltpu.async_copy(z_hbm.at[iib], zib, sg[1]),
              pltpu.async_copy(z_hbm.at[jjb], zjb, sg[1])]
        gp = [pltpu.async_copy(av[q].at[iib], s[6 + q], sg[2])
              for q in range(4)]
        gq = [pltpu.async_copy(av[q].at[jjb], s[10 + q], sg[2])
              for q in range(4)]
        return gz, gp + gq

    def pair_and_w(s, sg):
        zib, zjb, pairb, db, ebuf = s[2], s[3], s[4], s[5], s[14]
        for k in range(CH // L):
            sl = pl.ds(k * L, L)
            pairb[sl] = zib[sl] * MAX_Z + zjb[sl]
            s4i = plsc.load_gather(r4s_v, [zib[sl]])
            s4j = plsc.load_gather(r4s_v, [zjb[sl]])
            r = db[sl] * (1.0 / D3_AUTOANG) + 1e-6
            r2 = r * r
            r6 = r2 * r2 * r2
            r8 = r6 * r2
            ss = s4i * s4j
            qq = 3.0 * ss * ss
            r0 = (A1 * SQRT3) * ss + A2
            r02 = r0 * r0
            r06 = r02 * r02 * r02
            r08 = r06 * r02
            ebuf[sl] = (-0.5 * S6) / (r6 + r06) \
                + ((-0.5 * S8) * qq) / (r8 + r08)
        return pltpu.async_copy(c6_hbm.at[pairb], cb, sg[3])

    def c6_scale_scatter(s):
        iib, ebuf = s[0], s[14]
        for k in range(CH // L):
            sl = pl.ds(k * L, L)
            eids = k * L + lane
            pi = [s[6 + q][sl] for q in range(4)]
            pj = [s[10 + q][sl] for q in range(4)]
            pi.append(one16 - pi[0] - pi[1] - pi[2] - pi[3])
            pj.append(one16 - pj[0] - pj[1] - pj[2] - pj[3])
            c6 = jnp.zeros((L,), f32)
            for a in range(5):
                rowacc = jnp.zeros((L,), f32)
                for b in range(5):
                    cab = plsc.load_gather(
                        cb, [eids, jnp.full((L,), a * 5 + b, i32)])
                    rowacc = rowacc + pj[b] * cab
                c6 = c6 + pi[a] * rowacc
            ebuf[sl] = c6 * ebuf[sl]
        pltpu.sync_copy(ebuf, en_sh.at[iib], add=True)

    def half(c2, carry):
        baseA = ebase + (c2 * 2) * CH
        dA = issue_loads(baseA, setA, semA)
        dB = issue_loads(baseA + CH, setB, semB)
        for d in dA:
            d.wait()
        gzA, gpA = issue_gathers(setA, semA)
        for g in gzA:
            g.wait()
        g3A = pair_and_w(setA, semA)
        for d in dB:
            d.wait()
        gzB, gpB = issue_gathers(setB, semB)
        for g in gpA:
            g.wait()
        g3A.wait()
        c6_scale_scatter(setA)
        for g in gzB:
            g.wait()
        g3B = pair_and_w(setB, semB)
        for g in gpB:
            g.wait()
        g3B.wait()
        c6_scale_scatter(setB)
        return carry

    lax.fori_loop(0, NCH // 2, half, 0)
    plsc.subcore_barrier()
    pltpu.sync_copy(en_sh.at[pl.ds(sid * NPT, NPT)], tbuf)
    pltpu.sync_copy(tbuf, out_hbm.at[pl.ds(cid * NP + sid * NPT, NPT)])


@functools.partial(
    pl.kernel,
    out_type=jax.ShapeDtypeStruct((NP,), f32),
    mesh=_mesh,
    compiler_params=pltpu.CompilerParams(needs_layout_passes=False),
    scratch_types=[
        pltpu.VMEM((AP,), f32),
        pltpu.VMEM((AP,), f32),
        pltpu.SemaphoreType.DMA,
    ],
)
def _k4(part_hbm, out_hbm, s_v, tmp, sem):
    cid = lax.axis_index("c")
    sid = lax.axis_index("s")
    wid = sid * NC + cid
    abase = wid * AP
    pltpu.sync_copy(part_hbm.at[pl.ds(abase, AP)], s_v)
    pltpu.sync_copy(part_hbm.at[pl.ds(NP + abase, AP)], tmp)

    def add(i, c):
        sl = pl.ds(i * L, L)
        s_v[sl] = s_v[sl] + tmp[sl]
        return c

    lax.fori_loop(0, AP // L, add, 0)
    pltpu.sync_copy(s_v, out_hbm.at[pl.ds(abase, AP)])


def kernel(Z, edge_dist, edge_index, rcov, r4r2, cn_ref, c6_ref):
    Zp = jnp.concatenate([Z.astype(i32), jnp.zeros((NP - N,), i32)])
    ii = jnp.concatenate(
        [edge_index[0].astype(i32), jnp.full((EPT - E,), N, i32)])
    jj = jnp.concatenate(
        [edge_index[1].astype(i32), jnp.full((EPT - E,), N, i32)])
    dist = jnp.concatenate(
        [edge_dist.astype(f32), jnp.ones((EPT - E,), f32)])
    rcov96 = jnp.pad(rcov.astype(f32), (0, 96 - MAX_Z))
    r4s96 = jnp.pad(jnp.sqrt(r4r2.astype(f32)), (0, 96 - MAX_Z))
    cnref480 = jnp.pad(cn_ref.astype(f32).reshape(-1), (0, 5))
    c6p = jnp.pad(c6_ref.astype(f32).reshape(MAX_Z * MAX_Z, 25),
                  ((0, 0), (0, 103)))

    cnpart = _k1(Zp, rcov96, ii, jj, dist)
    p5 = _k2(cnpart, Zp, cnref480)
    enpart = _k3(p5[0], p5[1], p5[2], p5[3],
                 c6p, Zp, r4s96, ii, jj, dist)
    out = _k4(enpart)
    return out[:N]
